# baseline (speedup 1.0000x reference)
import jax
import jax.numpy as jnp
from jax import lax
from jax.experimental import pallas as pl
from jax.experimental.pallas import tpu as pltpu
from jax.experimental.pallas import tpu_sc as plsc

_B = 16384
_EMBED = 32
_NC = 2
_NS = 16
_NW = _NC * _NS
_BPW = _B // _NW
_CHUNK = 128
_NCHUNK = _BPW // _CHUNK

_TCOLS = 32768
_H1 = 64
_HGROUP = 128 // _H1


def _pack_body(xT_ref, w_ref, y_ref):
  res = lax.dot_general(xT_ref[...], w_ref[...], (((0,), (0,)), ((), ())),
                        preferred_element_type=jnp.float32)
  res4 = res.reshape(_TCOLS // 16, _HGROUP, 8, _H1)
  pieces = [res4[:, k].reshape(_TCOLS // _HGROUP, _H1)
            for k in range(_HGROUP)]
  y_ref[...] = jnp.concatenate(pieces, axis=1)


def _pack_h1(tabT, w1half, n_rows):
  grid = pl.cdiv(n_rows, _TCOLS)
  return pl.pallas_call(
      _pack_body,
      grid=(grid,),
      in_specs=[
          pl.BlockSpec((_EMBED, _TCOLS), lambda i: (0, i)),
          pl.BlockSpec((_EMBED, _H1), lambda i: (0, 0)),
      ],
      out_specs=pl.BlockSpec((_TCOLS // _HGROUP, 128), lambda i: (i, 0)),
      out_shape=jax.ShapeDtypeStruct((grid * (_TCOLS // _HGROUP), 128),
                                     jnp.float32),
  )(tabT, w1half)


def _packed_coords(idx):
  q = ((_TCOLS // _HGROUP) * (idx // _TCOLS)
       + 8 * ((idx // 16) % (_TCOLS // 16)) + idx % 8)
  rem = (idx // 8) % _HGROUP
  return q, rem


def _sc_gather_body(utab, mtab, uidx, midx, uemb, memb,
                    idx_v, rows_a, rows_b, sem_a, sem_b):
  wid = lax.axis_index("s") * _NC + lax.axis_index("c")
  base = wid * _BPW

  def gather_table(tab4, idx_hbm, out_hbm):
    pltpu.sync_copy(idx_hbm.at[pl.ds(base, _BPW)], idx_v)

    def fire(j, buf, sem):
      return pltpu.async_copy(
          tab4.at[idx_v.at[pl.ds(j * _CHUNK, _CHUNK)]], buf, sem)

    cp = fire(0, rows_a, sem_a)
    for j in range(_NCHUNK):
      nxt = None
      if j + 1 < _NCHUNK:
        nxt = fire(j + 1, rows_b if j % 2 == 0 else rows_a,
                   sem_b if j % 2 == 0 else sem_a)
      cp.wait()
      rows = rows_a if j % 2 == 0 else rows_b
      pltpu.sync_copy(rows, out_hbm.at[pl.ds(base + j * _CHUNK, _CHUNK)])
      cp = nxt

  gather_table(utab, uidx, uemb)
  gather_table(mtab, midx, memb)


def _sc_gather(utab4, mtab4, uq, mq):
  mesh = plsc.VectorSubcoreMesh(core_axis_name="c", subcore_axis_name="s")
  f = pl.kernel(
      _sc_gather_body,
      out_type=(
          jax.ShapeDtypeStruct((_B, 128), jnp.float32),
          jax.ShapeDtypeStruct((_B, 128), jnp.float32),
      ),
      mesh=mesh,
      compiler_params=pltpu.CompilerParams(use_tc_tiling_on_sc=True),
      scratch_types=[
          pltpu.VMEM((_BPW,), jnp.int32),
          pltpu.VMEM((_CHUNK, 128), jnp.float32),
          pltpu.VMEM((_CHUNK, 128), jnp.float32),
          pltpu.SemaphoreType.DMA,
          pltpu.SemaphoreType.DMA,
      ],
  )
  return f(utab4, mtab4, uq, mq)


def _select_sub(x4, rem):
  return jnp.where(rem == 0, x4[:, 0:_H1], x4[:, _H1:2 * _H1])


def _mlp_body(u4_ref, m4_ref, urem_ref, mrem_ref, b1_ref,
              w2_ref, b2_ref, w3_ref, b3_ref, o_ref):
  h = (_select_sub(u4_ref[...], urem_ref[...])
       + _select_sub(m4_ref[...], mrem_ref[...]) + b1_ref[...])
  h = jnp.maximum(h, 0.0)
  h = jnp.dot(h, w2_ref[...], preferred_element_type=jnp.float32) + b2_ref[...]
  h = jnp.maximum(h, 0.0)
  y = jnp.dot(h, w3_ref[...], preferred_element_type=jnp.float32) + b3_ref[...]
  o_ref[...] = jax.nn.sigmoid(y)


_MLP_BLOCK = 2048


def _mlp(u4, m4, urem, mrem, b1, w2, b2, w3, b3):
  grid = (_B // _MLP_BLOCK,)
  full = lambda shape: pl.BlockSpec(shape, lambda i: (0, 0))
  return pl.pallas_call(
      _mlp_body,
      grid=grid,
      in_specs=[
          pl.BlockSpec((_MLP_BLOCK, 128), lambda i: (i, 0)),
          pl.BlockSpec((_MLP_BLOCK, 128), lambda i: (i, 0)),
          pl.BlockSpec((_MLP_BLOCK, 1), lambda i: (i, 0)),
          pl.BlockSpec((_MLP_BLOCK, 1), lambda i: (i, 0)),
          full(b1.shape),
          full(w2.shape), full(b2.shape), full(w3.shape), full(b3.shape),
      ],
      out_specs=pl.BlockSpec((_MLP_BLOCK, 1), lambda i: (i, 0)),
      out_shape=jax.ShapeDtypeStruct((_B, 1), jnp.float32),
  )(u4, m4, urem, mrem, b1, w2, b2, w3, b3)


def kernel(user_idx, movie_idx, user_table, movie_table, W1, b1, W2, b2, W3, b3):
  user_idx = user_idx.astype(jnp.int32)
  movie_idx = movie_idx.astype(jnp.int32)
  utab4 = _pack_h1(user_table.T, W1[:_EMBED], user_table.shape[0])
  mtab4 = _pack_h1(movie_table.T, W1[_EMBED:], movie_table.shape[0])
  uq, urem = _packed_coords(user_idx)
  mq, mrem = _packed_coords(movie_idx)
  u4, m4 = _sc_gather(utab4, mtab4, uq, mq)
  y = _mlp(u4, m4,
           urem.reshape(_B, 1),
           mrem.reshape(_B, 1),
           b1.reshape(1, -1),
           W2, b2.reshape(1, -1), W3, b3.reshape(1, 1))
  return y


# speedup vs baseline: 1.0127x; 1.0127x over previous
import jax
import jax.numpy as jnp
from jax import lax
from jax.experimental import pallas as pl
from jax.experimental.pallas import tpu as pltpu
from jax.experimental.pallas import tpu_sc as plsc

_B = 16384
_EMBED = 32
_NC = 2
_NS = 16
_NW = _NC * _NS
_BPW = _B // _NW
_CHUNK = 128
_NCHUNK = _BPW // _CHUNK

_TCOLS = 16384
_H1 = 64
_HGROUP = 128 // _H1


def _pack_body(xT_ref, w_ref, y_ref):
  res = lax.dot_general(xT_ref[...], w_ref[...], (((0,), (0,)), ((), ())),
                        preferred_element_type=jnp.float32)
  res4 = res.reshape(_TCOLS // 16, _HGROUP, 8, _H1)
  pieces = [res4[:, k].reshape(_TCOLS // _HGROUP, _H1)
            for k in range(_HGROUP)]
  y_ref[...] = jnp.concatenate(pieces, axis=1)


def _pack_h1(tabT, w1half, n_rows):
  grid = pl.cdiv(n_rows, _TCOLS)
  return pl.pallas_call(
      _pack_body,
      grid=(grid,),
      in_specs=[
          pl.BlockSpec((_EMBED, _TCOLS), lambda i: (0, i)),
          pl.BlockSpec((_EMBED, _H1), lambda i: (0, 0)),
      ],
      out_specs=pl.BlockSpec((_TCOLS // _HGROUP, 128), lambda i: (i, 0)),
      out_shape=jax.ShapeDtypeStruct((grid * (_TCOLS // _HGROUP), 128),
                                     jnp.float32),
  )(tabT, w1half)


def _packed_coords(idx):
  q = ((_TCOLS // _HGROUP) * (idx // _TCOLS)
       + 8 * ((idx // 16) % (_TCOLS // 16)) + idx % 8)
  rem = (idx // 8) % _HGROUP
  return q, rem


def _sc_gather_body(utab, mtab, uidx, midx, uemb, memb,
                    idx_v, rows_a, rows_b, sem_a, sem_b):
  wid = lax.axis_index("s") * _NC + lax.axis_index("c")
  base = wid * _BPW

  def gather_table(tab4, idx_hbm, out_hbm):
    pltpu.sync_copy(idx_hbm.at[pl.ds(base, _BPW)], idx_v)

    def fire(j, buf, sem):
      return pltpu.async_copy(
          tab4.at[idx_v.at[pl.ds(j * _CHUNK, _CHUNK)]], buf, sem)

    cp = fire(0, rows_a, sem_a)
    for j in range(_NCHUNK):
      nxt = None
      if j + 1 < _NCHUNK:
        nxt = fire(j + 1, rows_b if j % 2 == 0 else rows_a,
                   sem_b if j % 2 == 0 else sem_a)
      cp.wait()
      rows = rows_a if j % 2 == 0 else rows_b
      pltpu.sync_copy(rows, out_hbm.at[pl.ds(base + j * _CHUNK, _CHUNK)])
      cp = nxt

  gather_table(utab, uidx, uemb)
  gather_table(mtab, midx, memb)


def _sc_gather(utab4, mtab4, uq, mq):
  mesh = plsc.VectorSubcoreMesh(core_axis_name="c", subcore_axis_name="s")
  f = pl.kernel(
      _sc_gather_body,
      out_type=(
          jax.ShapeDtypeStruct((_B, 128), jnp.float32),
          jax.ShapeDtypeStruct((_B, 128), jnp.float32),
      ),
      mesh=mesh,
      compiler_params=pltpu.CompilerParams(use_tc_tiling_on_sc=True),
      scratch_types=[
          pltpu.VMEM((_BPW,), jnp.int32),
          pltpu.VMEM((_CHUNK, 128), jnp.float32),
          pltpu.VMEM((_CHUNK, 128), jnp.float32),
          pltpu.SemaphoreType.DMA,
          pltpu.SemaphoreType.DMA,
      ],
  )
  return f(utab4, mtab4, uq, mq)


def _select_sub(x4, rem):
  return jnp.where(rem == 0, x4[:, 0:_H1], x4[:, _H1:2 * _H1])


def _mlp_body(u4_ref, m4_ref, urem_ref, mrem_ref, b1_ref,
              w2_ref, b2_ref, w3_ref, b3_ref, o_ref):
  h = (_select_sub(u4_ref[...], urem_ref[...])
       + _select_sub(m4_ref[...], mrem_ref[...]) + b1_ref[...])
  h = jnp.maximum(h, 0.0)
  h = jnp.dot(h, w2_ref[...], preferred_element_type=jnp.float32) + b2_ref[...]
  h = jnp.maximum(h, 0.0)
  y = jnp.dot(h, w3_ref[...], preferred_element_type=jnp.float32) + b3_ref[...]
  o_ref[...] = jax.nn.sigmoid(y)


_MLP_BLOCK = 2048


def _mlp(u4, m4, urem, mrem, b1, w2, b2, w3, b3):
  grid = (_B // _MLP_BLOCK,)
  full = lambda shape: pl.BlockSpec(shape, lambda i: (0, 0))
  return pl.pallas_call(
      _mlp_body,
      grid=grid,
      in_specs=[
          pl.BlockSpec((_MLP_BLOCK, 128), lambda i: (i, 0)),
          pl.BlockSpec((_MLP_BLOCK, 128), lambda i: (i, 0)),
          pl.BlockSpec((_MLP_BLOCK, 1), lambda i: (i, 0)),
          pl.BlockSpec((_MLP_BLOCK, 1), lambda i: (i, 0)),
          full(b1.shape),
          full(w2.shape), full(b2.shape), full(w3.shape), full(b3.shape),
      ],
      out_specs=pl.BlockSpec((_MLP_BLOCK, 1), lambda i: (i, 0)),
      out_shape=jax.ShapeDtypeStruct((_B, 1), jnp.float32),
  )(u4, m4, urem, mrem, b1, w2, b2, w3, b3)


def kernel(user_idx, movie_idx, user_table, movie_table, W1, b1, W2, b2, W3, b3):
  user_idx = user_idx.astype(jnp.int32)
  movie_idx = movie_idx.astype(jnp.int32)
  utab4 = _pack_h1(user_table.T, W1[:_EMBED], user_table.shape[0])
  mtab4 = _pack_h1(movie_table.T, W1[_EMBED:], movie_table.shape[0])
  uq, urem = _packed_coords(user_idx)
  mq, mrem = _packed_coords(movie_idx)
  u4, m4 = _sc_gather(utab4, mtab4, uq, mq)
  y = _mlp(u4, m4,
           urem.reshape(_B, 1),
           mrem.reshape(_B, 1),
           b1.reshape(1, -1),
           W2, b2.reshape(1, -1), W3, b3.reshape(1, 1))
  return y
